# baseline (speedup 1.0000x reference)
import functools

import jax
import jax.numpy as jnp
from jax import lax
from jax.experimental import pallas as pl
from jax.experimental.pallas import tpu as pltpu
from jax.experimental.pallas import tpu_sc as plsc


def _router_body(x_ref, wg_ref, flat_ref, wts_ref, meta_ref, aux_ref, *,
                 seq, e_num, nb, bt, rb):
    x = x_ref[...]
    wg = wg_ref[...]
    logits = jnp.dot(x, wg, preferred_element_type=jnp.float32)
    m = jnp.max(logits, axis=1, keepdims=True)
    ex = jnp.exp(logits - m)
    p = ex / jnp.sum(ex, axis=1, keepdims=True)
    lane = lax.broadcasted_iota(jnp.int32, p.shape, 1)
    m1 = jnp.max(p, axis=1, keepdims=True)
    a1 = jnp.min(jnp.where(p == m1, lane, e_num), axis=1, keepdims=True)
    oh1 = lane == a1
    p2 = jnp.where(oh1, -jnp.inf, p)
    m2 = jnp.max(p2, axis=1, keepdims=True)
    a2 = jnp.min(jnp.where(p2 == m2, lane, e_num), axis=1, keepdims=True)
    oh2 = lane == a2
    denom = m1 + m2
    w1 = m1 / denom
    w2 = m2 / denom
    onehot12 = jnp.where(oh1 | oh2, 1.0, 0.0)

    col = lax.broadcasted_iota(jnp.int32, (rb, seq), 1)
    pos_rows = []
    for r in range(seq // rb):
        row = lax.broadcasted_iota(jnp.int32, (rb, seq), 0) + r * rb
        tril = jnp.where(row > col, 1.0, 0.0)
        pos_rows.append(jnp.dot(tril, onehot12, preferred_element_type=jnp.float32))
    pos = jnp.concatenate(pos_rows, axis=0)
    counts = jnp.sum(onehot12, axis=0, keepdims=True)

    pos1 = jnp.sum(jnp.where(oh1, pos, 0.0), axis=1, keepdims=True)
    pos2 = jnp.sum(jnp.where(oh2, pos, 0.0), axis=1, keepdims=True)
    flat1 = a1.astype(jnp.float32) * seq + pos1
    flat2 = a2.astype(jnp.float32) * seq + pos2
    flat_ref[...] = jnp.concatenate(
        [flat1.T, flat2.T], axis=0).astype(jnp.int32)
    wts_ref[...] = jnp.concatenate([w1.T, w2.T], axis=0)

    nl = e_num * nb
    l1 = lax.broadcasted_iota(jnp.int32, (1, nl), 1)
    e_of = l1 // nb
    jb = (l1 - e_of * nb).astype(jnp.float32)
    ce = lax.broadcasted_iota(jnp.int32, (e_num, nl), 0)
    r_mat = jnp.where(ce == lax.broadcasted_iota(jnp.int32, (e_num, nl), 1) // nb,
                      1.0, 0.0)
    cbb = jnp.dot(counts, r_mat, preferred_element_type=jnp.float32)
    act = jnp.where(jb * bt < cbb, 1.0, 0.0)
    mxb = jnp.maximum(jnp.floor((cbb - 1.0) / bt), 0.0)
    bmap = e_of.astype(jnp.float32) * nb + jnp.minimum(jb, mxb)
    cpad = jnp.concatenate(
        [counts, jnp.zeros((1, nl - e_num), jnp.float32)], axis=1)
    meta_ref[...] = jnp.concatenate([cpad, bmap, act], axis=0).astype(jnp.int32)
    aux_ref[0, 0] = jnp.sum(jnp.mean(p, axis=0) ** 2) * e_num


def _scatter_body(flat_hbm, zeros_hbm, tl_hbm, tl_v, flat_v, *, seq, npair):
    cid = lax.axis_index("c")
    sid = lax.axis_index("s")

    @pl.when((cid == 0) & (sid == 0))
    def _():
        pltpu.sync_copy(zeros_hbm, tl_v)
        pltpu.sync_copy(flat_hbm, flat_v)
        per_slot = seq // 16
        for j in range(npair // 16):
            tok0 = (j % per_slot) * 16
            idx = flat_v[pl.ds(j * 16, 16)]
            vals = lax.iota(jnp.int32, 16) + tok0
            plsc.store_scatter(tl_v, [idx], vals)
        pltpu.sync_copy(tl_v, tl_hbm)


def _gather_body(xf_hbm, tl_hbm, cnt_hbm, xg_hbm, cnt_v, idx_v, rows_v, sem,
                 *, seq, nsub, ch):
    cid = lax.axis_index("c")
    sid = lax.axis_index("s")
    wid = sid * 2 + cid
    qper = 32 // nsub
    e = wid // qper
    q = wid - e * qper
    qlen = seq // qper
    pltpu.sync_copy(cnt_hbm, cnt_v)
    lane16 = lax.iota(jnp.int32, 16)
    cnt = jnp.sum(jnp.where(lane16 == e, cnt_v[...], 0))
    act = jnp.clip(cnt - q * qlen, 0, qlen)
    nch = (act + ch - 1) // ch
    base = e * seq + q * qlen

    def body(ci, carry):
        start = base + ci * ch
        pltpu.sync_copy(tl_hbm.at[pl.ds(start, ch)], idx_v)
        pltpu.async_copy(xf_hbm.at[idx_v], rows_v, sem).wait()
        pltpu.sync_copy(rows_v, xg_hbm.at[pl.ds(start, ch)])
        return carry

    lax.fori_loop(0, nch, body, 0)


def _expert_body(bm_ref, ba_ref, xg_ref, w1_ref, w2_ref, yg_ref):
    j = pl.program_id(0)

    @pl.when(ba_ref[j] != 0)
    def _():
        x = xg_ref[...].astype(jnp.bfloat16)
        h = jnp.dot(x, w1_ref[0], preferred_element_type=jnp.float32)
        h = h * jax.nn.sigmoid(h)
        y = jnp.dot(h.astype(jnp.bfloat16), w2_ref[0],
                    preferred_element_type=jnp.float32)
        yg_ref[...] = y


def _shared_body(x_ref, w1_ref, w2_ref, out_ref, *, bt, ntb):
    k = pl.program_id(0)
    for r in range(ntb):
        row = r * bt
        x = x_ref[pl.ds(row, bt), :]
        h = jnp.dot(x, w1_ref[0], preferred_element_type=jnp.float32)
        h = h * jax.nn.sigmoid(h)
        y = jnp.dot(h.astype(jnp.bfloat16), w2_ref[0],
                    preferred_element_type=jnp.float32)

        @pl.when(k == 0)
        def _init():
            out_ref[pl.ds(row, bt), :] = y

        @pl.when(k > 0)
        def _acc():
            out_ref[pl.ds(row, bt), :] += y


def _combine_body(sh_hbm, yg_hbm, flat_hbm, wts_hbm, out_hbm,
                  acc_v, rows_v, f_v, w_v, sem, *, seq, dim, ch):
    cid = lax.axis_index("c")
    sid = lax.axis_index("s")
    wid = sid * 2 + cid
    tper = seq // 32
    base = wid * tper
    ng = dim // 16
    for c in range(tper // ch):
        t0 = base + c * ch
        pltpu.sync_copy(sh_hbm.at[pl.ds(t0, ch)], acc_v)
        for slot in range(2):
            pltpu.sync_copy(flat_hbm.at[slot, pl.ds(t0, ch)], f_v)
            pltpu.sync_copy(wts_hbm.at[slot, pl.ds(t0, ch)], w_v)
            pltpu.async_copy(yg_hbm.at[f_v], rows_v, sem).wait()
            wvec = w_v[...]
            for t in range(ch):
                wsc = wvec[t]

                def gbody(g, carry, t=t, wsc=wsc):
                    off = g * 16
                    acc_v[t, pl.ds(off, 16)] += wsc * rows_v[t, pl.ds(off, 16)]
                    return carry

                lax.fori_loop(0, ng, gbody, 0)
        pltpu.sync_copy(acc_v, out_hbm.at[pl.ds(t0, ch)])


def kernel(x, Wg, Ws1, Ws2, We1, We2):
    orig_shape = x.shape
    dim = orig_shape[-1]
    xf = x.reshape(-1, dim)
    seq = xf.shape[0]
    e_num = Wg.shape[1]
    nsh = Ws1.shape[0]
    hid = Ws1.shape[2]
    bt = 256
    nb = seq // bt
    nblk = e_num * nb

    flat2, wts2, meta, aux = pl.pallas_call(
        functools.partial(_router_body, seq=seq, e_num=e_num, nb=nb, bt=bt,
                          rb=256),
        out_shape=[
            jax.ShapeDtypeStruct((2, seq), jnp.int32),
            jax.ShapeDtypeStruct((2, seq), jnp.float32),
            jax.ShapeDtypeStruct((3, nblk), jnp.int32),
            jax.ShapeDtypeStruct((1, 1), jnp.float32),
        ],
        out_specs=[
            pl.BlockSpec(memory_space=pltpu.VMEM),
            pl.BlockSpec(memory_space=pltpu.VMEM),
            pl.BlockSpec(memory_space=pltpu.VMEM),
            pl.BlockSpec(memory_space=pltpu.SMEM),
        ],
        compiler_params=pltpu.CompilerParams(
            vmem_limit_bytes=100 * 1024 * 1024,
        ),
    )(xf, Wg)

    counts16 = meta[0, :16]
    cnts = counts16[:e_num]
    e_of_h = jnp.arange(nblk, dtype=jnp.int32) // nb
    jb_h = jnp.arange(nblk, dtype=jnp.int32) % nb
    ce_h = cnts[e_of_h]
    ba = (jb_h * bt < ce_h).astype(jnp.int32)
    last_h = jnp.maximum((ce_h - 1) // bt, 0)
    bm = (e_of_h * nb + jnp.minimum(jb_h, last_h)).astype(jnp.int32)

    mesh = plsc.VectorSubcoreMesh(core_axis_name="c", subcore_axis_name="s")
    tl_len = e_num * seq

    scat = functools.partial(
        pl.kernel,
        mesh=mesh,
        compiler_params=pltpu.CompilerParams(needs_layout_passes=False),
        out_type=jax.ShapeDtypeStruct((tl_len,), jnp.int32),
        scratch_types=[
            pltpu.VMEM((tl_len,), jnp.int32),
            pltpu.VMEM((2 * seq,), jnp.int32),
        ],
    )(functools.partial(_scatter_body, seq=seq, npair=2 * seq))
    token_list = scat(flat2.reshape(2 * seq), jnp.zeros((tl_len,), jnp.int32))

    ch = 32
    gath = functools.partial(
        pl.kernel,
        mesh=mesh,
        compiler_params=pltpu.CompilerParams(needs_layout_passes=False),
        out_type=jax.ShapeDtypeStruct((tl_len, dim), jnp.float32),
        scratch_types=[
            pltpu.VMEM((16,), jnp.int32),
            pltpu.VMEM((ch,), jnp.int32),
            pltpu.VMEM((ch, dim), jnp.float32),
            pltpu.SemaphoreType.DMA,
        ],
    )(functools.partial(_gather_body, seq=seq, nsub=e_num, ch=ch))
    xg = gath(xf, token_list, counts16)

    hc = 384
    hid_pad = ((hid + hc - 1) // hc) * hc
    w1r, w2r, w1s, w2s = We1, We2, Ws1, Ws2
    if hid_pad != hid:
        w1r = jnp.pad(w1r, ((0, 0), (0, 0), (0, hid_pad - hid)))
        w2r = jnp.pad(w2r, ((0, 0), (0, hid_pad - hid), (0, 0)))
        w1s = jnp.pad(w1s, ((0, 0), (0, 0), (0, hid_pad - hid)))
        w2s = jnp.pad(w2s, ((0, 0), (0, hid_pad - hid), (0, 0)))
    w1r = w1r.astype(jnp.bfloat16)
    w2r = w2r.astype(jnp.bfloat16)
    w1s = w1s.astype(jnp.bfloat16)
    w2s = w2s.astype(jnp.bfloat16)
    hp = hid_pad

    grid_spec = pltpu.PrefetchScalarGridSpec(
        num_scalar_prefetch=2,
        grid=(nblk,),
        in_specs=[
            pl.BlockSpec((bt, dim), lambda j, bm_r, ba_r: (bm_r[j], 0)),
            pl.BlockSpec((1, dim, hp), lambda j, bm_r, ba_r, nb=nb: (j // nb, 0, 0)),
            pl.BlockSpec((1, hp, dim), lambda j, bm_r, ba_r, nb=nb: (j // nb, 0, 0)),
        ],
        out_specs=pl.BlockSpec((bt, dim), lambda j, bm_r, ba_r: (bm_r[j], 0)),
    )
    yg = pl.pallas_call(
        _expert_body,
        grid_spec=grid_spec,
        out_shape=jax.ShapeDtypeStruct((tl_len, dim), jnp.float32),
        compiler_params=pltpu.CompilerParams(
            dimension_semantics=("arbitrary",),
            vmem_limit_bytes=100 * 1024 * 1024,
        ),
    )(bm, ba, xg, w1r, w2r)

    xbf = xf.astype(jnp.bfloat16)
    sh = pl.pallas_call(
        functools.partial(_shared_body, bt=bt, ntb=seq // bt),
        grid=(nsh,),
        in_specs=[
            pl.BlockSpec((seq, dim), lambda k: (0, 0)),
            pl.BlockSpec((1, dim, hp), lambda k: (k, 0, 0)),
            pl.BlockSpec((1, hp, dim), lambda k: (k, 0, 0)),
        ],
        out_specs=pl.BlockSpec((seq, dim), lambda k: (0, 0)),
        out_shape=jax.ShapeDtypeStruct((seq, dim), jnp.float32),
        compiler_params=pltpu.CompilerParams(
            dimension_semantics=("arbitrary",),
            vmem_limit_bytes=100 * 1024 * 1024,
        ),
    )(xbf, w1s, w2s)

    ch4 = 16
    comb = functools.partial(
        pl.kernel,
        mesh=mesh,
        compiler_params=pltpu.CompilerParams(needs_layout_passes=False),
        out_type=jax.ShapeDtypeStruct((seq, dim), jnp.float32),
        scratch_types=[
            pltpu.VMEM((ch4, dim), jnp.float32),
            pltpu.VMEM((ch4, dim), jnp.float32),
            pltpu.VMEM((ch4,), jnp.int32),
            pltpu.VMEM((ch4,), jnp.float32),
            pltpu.SemaphoreType.DMA,
        ],
    )(functools.partial(_combine_body, seq=seq, dim=dim, ch=ch4))
    out = comb(sh, yg, flat2, wts2)

    return out.reshape(orig_shape), aux[0, 0]
